# baseline (speedup 1.0000x reference)
import functools

import jax
import jax.numpy as jnp
from jax import lax
from jax.experimental import pallas as pl
from jax.experimental.pallas import tpu as pltpu
from jax.experimental.pallas import tpu_sc as plsc

D_IN = 1024
D_SAE = 32768
K = 64
TILE = 1024

NLANE = 16
NVREG = D_SAE // NLANE
CMAX = 2048
NSTR = 4
CAPS = 64
ECAPS = 192


G = D_SAE // TILE
NTHR = 65
NTHRP = 80
LADDER = [2.0 ** (-8.0 + 0.25 * j) for j in range(NTHR)]


def _enc_body(x_ref, bpre_ref, w_ref, benc_ref, z_ref, max_ref):
    xm = x_ref[...] - bpre_ref[...]
    z = jnp.dot(xm, w_ref[...], preferred_element_type=jnp.float32)
    z = jnp.maximum(z + benc_ref[...], 0.0)
    z_ref[...] = z
    i = pl.program_id(0)
    mx = jnp.max(z, axis=1)
    max_ref[:, pl.ds(i, 1), :] = jnp.broadcast_to(mx[:, None, None],
                                                  (32, 1, NLANE))


def _encode(x, b_pre, W_enc, b_enc):
    return pl.pallas_call(
        _enc_body,
        grid=(G,),
        in_specs=[
            pl.BlockSpec((32, D_IN), lambda i: (0, 0)),
            pl.BlockSpec((1, D_IN), lambda i: (0, 0)),
            pl.BlockSpec((D_IN, TILE), lambda i: (0, i)),
            pl.BlockSpec((1, TILE), lambda i: (0, i)),
        ],
        out_specs=[
            pl.BlockSpec((32, TILE), lambda i: (0, i)),
            pl.BlockSpec((32, G, NLANE), lambda i: (0, 0, 0)),
        ],
        out_shape=[
            jax.ShapeDtypeStruct((32, D_SAE), jnp.float32),
            jax.ShapeDtypeStruct((32, G, NLANE), jnp.float32),
        ],
    )(x, b_pre[None], W_enc, b_enc[None])


def _splat(v):
    return jnp.full((NLANE,), v, jnp.float32)


def _mid(lo, hi):
    return lo + 0.5 * (hi - lo)


def _sc_body(z_hbm, max_hbm, wdec_hbm, bpre_hbm,
             zsp_hbm, xhat_hbm,
             zrow, cslot, eidx, fvalp, fidxp, fidx, wrows, bprev,
             xrow, svb, maxb, sem_g, sem_z):
    c = lax.axis_index("c")
    s = lax.axis_index("s")
    r = s * 2 + c

    pltpu.sync_copy(z_hbm.at[r], zrow)
    pltpu.sync_copy(bpre_hbm, bprev)
    pltpu.sync_copy(max_hbm.at[r], maxb)

    iota16 = lax.iota(jnp.int32, NLANE)
    zero16f = jnp.zeros((NLANE,), jnp.float32)
    zero16i = jnp.zeros((NLANE,), jnp.int32)

    def _loads8(j8):
        return [zrow[pl.ds((j8 * 8 + u) * NLANE, NLANE)] for u in range(8)]

    def count_full(t):
        tf = _splat(t)

        def b(j8, cv):
            vs = _loads8(j8)
            ms = [(v > tf).astype(jnp.int32) for v in vs]
            s01 = ms[0] + ms[1]
            s23 = ms[2] + ms[3]
            s45 = ms[4] + ms[5]
            s67 = ms[6] + ms[7]
            return cv + ((s01 + s23) + (s45 + s67))

        cv = plsc.parallel_loop(0, NVREG // 8, carry=zero16i)(b)
        return jnp.sum(cv)

    mvx = zero16f
    for t_ in range(G):
        mvx = jnp.maximum(mvx, maxb[t_, pl.ds(0, NLANE)])
    M = jnp.max(mvx)
    tau0 = 0.5 * M
    c0 = count_full(tau0)
    hi0 = M


    def coarse_cond(st):
        lo, hi, tau, cnt = st
        in_range = jnp.logical_and(cnt >= K, cnt <= CMAX)
        mid = _mid(lo, hi)
        open_iv = jnp.logical_and(mid != lo, mid != hi)
        return jnp.logical_and(jnp.logical_not(in_range), open_iv)

    def coarse_body(st):
        lo, hi, tau, cnt = st
        lo = jnp.where(cnt > CMAX, tau, lo)
        hi = jnp.where(cnt < K, tau, hi)
        ntau = _mid(lo, hi)
        return (lo, hi, ntau, count_full(ntau))

    st = (jnp.float32(-1.0), hi0, tau0, c0)
    lo, hi, tau, cnt = lax.while_loop(coarse_cond, coarse_body, st)
    in_range = jnp.logical_and(cnt >= K, cnt <= CMAX)
    tauf = _splat(tau)

    pvec = [iota16 + p * NLANE for p in range(NSTR)]

    def compact_body(j8, cnts_):
        cs = list(cnts_)
        vs = _loads8(j8)
        for r in range(2):
            masks = []
            dsts = []
            for p in range(NSTR):
                v = vs[r * NSTR + p]
                masks.append(jnp.logical_and(v > tauf, cs[p] < CAPS))
                dsts.append(cs[p] * (NSTR * NLANE) + pvec[p])
            for p in range(NSTR):
                plsc.store_scatter(cslot, [dsts[p]], vs[r * NSTR + p],
                                   mask=masks[p])
            for p in range(NSTR):
                cs[p] = cs[p] + masks[p].astype(jnp.int32)
        return tuple(cs)

    cnts = plsc.parallel_loop(0, NVREG // 8,
                              carry=(zero16i,) * NSTR)(compact_body)
    maxc = jnp.max(jnp.maximum(jnp.maximum(cnts[0], cnts[1]),
                               jnp.maximum(cnts[2], cnts[3])))
    use_cand = jnp.logical_and(in_range, maxc <= CAPS)
    nblk = jnp.minimum(maxc, CAPS)

    def count_cand(t):
        tf2 = _splat(t)

        def b(i, cv):
            for k in range(NSTR):
                v = cslot[pl.ds((i * NSTR + k) * NLANE, NLANE)]
                ok = jnp.logical_and(v > tf2, cnts[k] > i)
                cv = cv + ok.astype(jnp.int32)
            return cv

        cv = plsc.parallel_loop(0, nblk, carry=zero16i)(b)
        return jnp.sum(cv)

    def count_sel(t):
        return lax.cond(use_cand, lambda: count_cand(t),
                        lambda: count_full(t))

    lo2 = jnp.where(in_range, tau, hi)
    hi2 = hi

    def fine_cond(st):
        lo_, hi_ = st
        mid = _mid(lo_, hi_)
        return jnp.logical_and(mid != lo_, mid != hi_)

    def fine_body(st):
        lo_, hi_ = st
        mid = _mid(lo_, hi_)
        cm = count_sel(mid)
        lo_ = jnp.where(cm >= K, mid, lo_)
        hi_ = jnp.where(cm >= K, hi_, mid)
        return (lo_, hi_)

    _, t = lax.while_loop(fine_cond, fine_body, (lo2, hi2))
    tf = _splat(t)
    cnt_gt = count_sel(t)
    m_eq = K - cnt_gt

    def ext_body(j8, ecnts_):
        es = list(ecnts_)
        vs = _loads8(j8)
        for r in range(2):
            masks = []
            dsts = []
            for p in range(NSTR):
                v = vs[r * NSTR + p]
                gt = v > tf
                eq = jnp.logical_and(v == tf, es[p] < 128)
                masks.append(jnp.logical_or(gt, eq))
                dsts.append(es[p] * (NSTR * NLANE) + pvec[p])
            for p in range(NSTR):
                j = j8 * 8 + r * NSTR + p
                plsc.store_scatter(eidx, [dsts[p]], iota16 + j * NLANE,
                                   mask=masks[p])
            for p in range(NSTR):
                es[p] = es[p] + masks[p].astype(jnp.int32)
        return tuple(es)

    ecnts = plsc.parallel_loop(0, NVREG // 8,
                               carry=(zero16i,) * NSTR)(ext_body)
    emax = jnp.max(jnp.maximum(jnp.maximum(ecnts[0], ecnts[1]),
                               jnp.maximum(ecnts[2], ecnts[3])))
    neblk = jnp.minimum(emax, ECAPS)

    def count_eq_le(ithr):
        it = jnp.full((NLANE,), ithr, jnp.int32)

        def b(i, cv):
            for k in range(NSTR):
                valid = ecnts[k] > i
                ei = eidx[pl.ds((i * NSTR + k) * NLANE, NLANE)]
                ev = plsc.load_gather(zrow, [ei], mask=valid)
                ok = jnp.logical_and(ev == tf, valid)
                ok = jnp.logical_and(ok, ei <= it)
                cv = cv + ok.astype(jnp.int32)
            return cv

        cv = lax.fori_loop(0, neblk, b, zero16i)
        return jnp.sum(cv)

    def eq_cond(st):
        lo_, hi_ = st
        return hi_ - lo_ > 1

    def eq_body(st):
        lo_, hi_ = st
        mid = lax.div(lo_ + hi_, 2)
        ce = count_eq_le(mid)
        lo_ = jnp.where(ce < m_eq, mid, lo_)
        hi_ = jnp.where(ce < m_eq, hi_, mid)
        return (lo_, hi_)

    _, ithr = lax.while_loop(eq_cond, eq_body,
                             (jnp.int32(-1), jnp.int32(D_SAE)))
    itf = jnp.full((NLANE,), ithr, jnp.int32)

    def fc_body(i, ptr):
        for k in range(NSTR):
            valid = ecnts[k] > i
            ei = eidx[pl.ds((i * NSTR + k) * NLANE, NLANE)]
            ev = plsc.load_gather(zrow, [ei], mask=valid)
            gtm = jnp.logical_and(ev > tf, valid)
            eqm = jnp.logical_and(jnp.logical_and(ev == tf, valid),
                                  ei <= itf)
            m = jnp.logical_or(gtm, eqm)
            plsc.store_compressed(fvalp.at[pl.ds(ptr, NLANE)], ev, mask=m)
            plsc.store_compressed(fidxp.at[pl.ds(ptr, NLANE)], ei, mask=m)
            ptr = ptr + plsc.all_reduce_population_count(m)[0]
        return ptr

    lax.fori_loop(0, neblk, fc_body, jnp.int32(0))

    for g in range(K // NLANE):
        fidx[pl.ds(g * NLANE, NLANE)] = fidxp[pl.ds(g * NLANE, NLANE)]
    gather = pltpu.async_copy(wdec_hbm.at[fidx], wrows, sem_g)

    def zz_body(j, _):
        zrow[pl.ds(j * NLANE, NLANE)] = zero16f
        return 0

    plsc.parallel_loop(0, NVREG // 8, carry=jnp.int32(0))(
        lambda j8, _: ([zrow.__setitem__(pl.ds((j8 * 8 + u) * NLANE, NLANE),
                                         zero16f) for u in range(8)], 0)[1])
    for g in range(K // NLANE):
        idxv = fidx[pl.ds(g * NLANE, NLANE)]
        valv = fvalp[pl.ds(g * NLANE, NLANE)]
        plsc.store_scatter(zrow, [idxv], valv)
    zout = pltpu.async_copy(zrow, zsp_hbm.at[r], sem_z)

    gather.wait()

    for g in range(K // NLANE):
        vv = fvalp[pl.ds(g * NLANE, NLANE)]
        for u in range(NLANE):
            svb[pl.ds((g * NLANE + u) * NLANE, NLANE)] = jnp.full(
                (NLANE,), vv[u])

    NCH = 8
    NU = D_IN // NCH // NLANE
    for ch in range(NCH):
        base = ch * (D_IN // NCH)

        def jbody(j2, accs):
            out = list(accs)
            for rr in range(2):
                j = j2 * 2 + rr
                sv = svb[pl.ds(j * NLANE, NLANE)]
                rows = [wrows[j, pl.ds(base + u * NLANE, NLANE)]
                        for u in range(NU)]
                out = [a + sv * rv for a, rv in zip(out, rows)]
            return tuple(out)

        accs = plsc.parallel_loop(0, K // 2, carry=(zero16f,) * NU)(jbody)
        for u, a in enumerate(accs):
            off = base + u * NLANE
            xrow[pl.ds(off, NLANE)] = a + bprev[pl.ds(off, NLANE)]

    pltpu.sync_copy(xrow, xhat_hbm.at[r])
    zout.wait()


def _topk_decode(z, maxs, W_dec, b_pre):
    mesh = plsc.VectorSubcoreMesh(core_axis_name="c", subcore_axis_name="s")
    f32 = jnp.float32
    kfn = functools.partial(
        pl.kernel,
        out_type=(
            jax.ShapeDtypeStruct((32, D_SAE), f32),
            jax.ShapeDtypeStruct((32, D_IN), f32),
        ),
        mesh=mesh,
        compiler_params=pltpu.CompilerParams(needs_layout_passes=False),
        scratch_types=[
            pltpu.VMEM((D_SAE,), f32),
            pltpu.VMEM((CAPS * NSTR * NLANE,), f32),
            pltpu.VMEM((ECAPS * NSTR * NLANE,), jnp.int32),
            pltpu.VMEM((K + NLANE,), f32),
            pltpu.VMEM((K + NLANE,), jnp.int32),
            pltpu.VMEM((K,), jnp.int32),
            pltpu.VMEM((K, D_IN), f32),
            pltpu.VMEM((D_IN,), f32),
            pltpu.VMEM((D_IN,), f32),
            pltpu.VMEM((K * NLANE,), f32),
            pltpu.VMEM((G, NLANE), f32),
            pltpu.SemaphoreType.DMA,
            pltpu.SemaphoreType.DMA,
        ],
    )(_sc_body)
    return kfn(z, maxs, W_dec, b_pre)


def kernel(x, b_pre, W_enc, b_enc, W_dec):
    z, maxs = _encode(x, b_pre, W_enc, b_enc)
    z_sparse, x_hat = _topk_decode(z, maxs, W_dec, b_pre)
    return (x_hat, z_sparse)


# speedup vs baseline: 1.0797x; 1.0797x over previous
import functools

import jax
import jax.numpy as jnp
from jax import lax
from jax.experimental import pallas as pl
from jax.experimental.pallas import tpu as pltpu
from jax.experimental.pallas import tpu_sc as plsc

D_IN = 1024
D_SAE = 32768
K = 64
TILE = 2048

NLANE = 16
NVREG = D_SAE // NLANE
CMAX = 2048
NSTR = 4
CAPS = 64
ECAPS = 192


G = D_SAE // TILE
NTHR = 65
NTHRP = 80
LADDER = [2.0 ** (-8.0 + 0.25 * j) for j in range(NTHR)]


def _enc_body(x_ref, bpre_ref, w_ref, benc_ref, z_ref, max_ref):
    xm = x_ref[...] - bpre_ref[...]
    z = jnp.dot(xm, w_ref[...], preferred_element_type=jnp.float32)
    z = jnp.maximum(z + benc_ref[...], 0.0)
    z_ref[...] = z
    i = pl.program_id(0)
    mx = jnp.max(z, axis=1)
    max_ref[:, pl.ds(i, 1), :] = jnp.broadcast_to(mx[:, None, None],
                                                  (32, 1, NLANE))


def _encode(x, b_pre, W_enc, b_enc):
    return pl.pallas_call(
        _enc_body,
        grid=(G,),
        in_specs=[
            pl.BlockSpec((32, D_IN), lambda i: (0, 0)),
            pl.BlockSpec((1, D_IN), lambda i: (0, 0)),
            pl.BlockSpec((D_IN, TILE), lambda i: (0, i)),
            pl.BlockSpec((1, TILE), lambda i: (0, i)),
        ],
        out_specs=[
            pl.BlockSpec((32, TILE), lambda i: (0, i)),
            pl.BlockSpec((32, G, NLANE), lambda i: (0, 0, 0)),
        ],
        out_shape=[
            jax.ShapeDtypeStruct((32, D_SAE), jnp.float32),
            jax.ShapeDtypeStruct((32, G, NLANE), jnp.float32),
        ],
    )(x, b_pre[None], W_enc, b_enc[None])


def _splat(v):
    return jnp.full((NLANE,), v, jnp.float32)


def _mid(lo, hi):
    return lo + 0.5 * (hi - lo)


def _sc_body(z_hbm, max_hbm, wdec_hbm, bpre_hbm,
             zsp_hbm, xhat_hbm,
             zrow, cslot, eidx, fvalp, fidxp, fidx, wrows, bprev,
             xrow, svb, maxb, sem_g, sem_z):
    c = lax.axis_index("c")
    s = lax.axis_index("s")
    r = s * 2 + c

    pltpu.sync_copy(z_hbm.at[r], zrow)
    pltpu.sync_copy(bpre_hbm, bprev)
    pltpu.sync_copy(max_hbm.at[r], maxb)

    iota16 = lax.iota(jnp.int32, NLANE)
    zero16f = jnp.zeros((NLANE,), jnp.float32)
    zero16i = jnp.zeros((NLANE,), jnp.int32)

    def _loads8(j8):
        return [zrow[pl.ds((j8 * 8 + u) * NLANE, NLANE)] for u in range(8)]

    def count_full(t):
        tf = _splat(t)

        def b(j8, cv):
            vs = _loads8(j8)
            ms = [(v > tf).astype(jnp.int32) for v in vs]
            s01 = ms[0] + ms[1]
            s23 = ms[2] + ms[3]
            s45 = ms[4] + ms[5]
            s67 = ms[6] + ms[7]
            return cv + ((s01 + s23) + (s45 + s67))

        cv = plsc.parallel_loop(0, NVREG // 8, carry=zero16i)(b)
        return jnp.sum(cv)

    mvx = zero16f
    for t_ in range(G):
        mvx = jnp.maximum(mvx, maxb[t_, pl.ds(0, NLANE)])
    M = jnp.max(mvx)
    tau0 = 0.5 * M
    c0 = count_full(tau0)
    hi0 = M


    def coarse_cond(st):
        lo, hi, tau, cnt = st
        in_range = jnp.logical_and(cnt >= K, cnt <= CMAX)
        mid = _mid(lo, hi)
        open_iv = jnp.logical_and(mid != lo, mid != hi)
        return jnp.logical_and(jnp.logical_not(in_range), open_iv)

    def coarse_body(st):
        lo, hi, tau, cnt = st
        lo = jnp.where(cnt > CMAX, tau, lo)
        hi = jnp.where(cnt < K, tau, hi)
        ntau = _mid(lo, hi)
        return (lo, hi, ntau, count_full(ntau))

    st = (jnp.float32(-1.0), hi0, tau0, c0)
    lo, hi, tau, cnt = lax.while_loop(coarse_cond, coarse_body, st)
    in_range = jnp.logical_and(cnt >= K, cnt <= CMAX)
    tauf = _splat(tau)

    pvec = [iota16 + p * NLANE for p in range(NSTR)]

    def compact_body(j8, cnts_):
        cs = list(cnts_)
        vs = _loads8(j8)
        for r in range(2):
            masks = []
            dsts = []
            for p in range(NSTR):
                v = vs[r * NSTR + p]
                masks.append(jnp.logical_and(v > tauf, cs[p] < CAPS))
                dsts.append(cs[p] * (NSTR * NLANE) + pvec[p])
            for p in range(NSTR):
                plsc.store_scatter(cslot, [dsts[p]], vs[r * NSTR + p],
                                   mask=masks[p])
            for p in range(NSTR):
                cs[p] = cs[p] + masks[p].astype(jnp.int32)
        return tuple(cs)

    cnts = plsc.parallel_loop(0, NVREG // 8,
                              carry=(zero16i,) * NSTR)(compact_body)
    maxc = jnp.max(jnp.maximum(jnp.maximum(cnts[0], cnts[1]),
                               jnp.maximum(cnts[2], cnts[3])))
    use_cand = jnp.logical_and(in_range, maxc <= CAPS)
    nblk = jnp.minimum(maxc, CAPS)

    def count_cand(t):
        tf2 = _splat(t)

        def b(i, cv):
            for k in range(NSTR):
                v = cslot[pl.ds((i * NSTR + k) * NLANE, NLANE)]
                ok = jnp.logical_and(v > tf2, cnts[k] > i)
                cv = cv + ok.astype(jnp.int32)
            return cv

        cv = plsc.parallel_loop(0, nblk, carry=zero16i)(b)
        return jnp.sum(cv)

    def count_sel(t):
        return lax.cond(use_cand, lambda: count_cand(t),
                        lambda: count_full(t))

    lo2 = jnp.where(in_range, tau, hi)
    hi2 = hi

    def fine_cond(st):
        lo_, hi_ = st
        mid = _mid(lo_, hi_)
        return jnp.logical_and(mid != lo_, mid != hi_)

    def fine_body(st):
        lo_, hi_ = st
        mid = _mid(lo_, hi_)
        cm = count_sel(mid)
        lo_ = jnp.where(cm >= K, mid, lo_)
        hi_ = jnp.where(cm >= K, hi_, mid)
        return (lo_, hi_)

    _, t = lax.while_loop(fine_cond, fine_body, (lo2, hi2))
    tf = _splat(t)
    cnt_gt = count_sel(t)
    m_eq = K - cnt_gt

    def ext_body(j8, ecnts_):
        es = list(ecnts_)
        vs = _loads8(j8)
        for r in range(2):
            masks = []
            dsts = []
            for p in range(NSTR):
                v = vs[r * NSTR + p]
                gt = v > tf
                eq = jnp.logical_and(v == tf, es[p] < 128)
                masks.append(jnp.logical_or(gt, eq))
                dsts.append(es[p] * (NSTR * NLANE) + pvec[p])
            for p in range(NSTR):
                j = j8 * 8 + r * NSTR + p
                plsc.store_scatter(eidx, [dsts[p]], iota16 + j * NLANE,
                                   mask=masks[p])
            for p in range(NSTR):
                es[p] = es[p] + masks[p].astype(jnp.int32)
        return tuple(es)

    ecnts = plsc.parallel_loop(0, NVREG // 8,
                               carry=(zero16i,) * NSTR)(ext_body)
    emax = jnp.max(jnp.maximum(jnp.maximum(ecnts[0], ecnts[1]),
                               jnp.maximum(ecnts[2], ecnts[3])))
    neblk = jnp.minimum(emax, ECAPS)

    def count_eq_le(ithr):
        it = jnp.full((NLANE,), ithr, jnp.int32)

        def b(i, cv):
            for k in range(NSTR):
                valid = ecnts[k] > i
                ei = eidx[pl.ds((i * NSTR + k) * NLANE, NLANE)]
                ev = plsc.load_gather(zrow, [ei], mask=valid)
                ok = jnp.logical_and(ev == tf, valid)
                ok = jnp.logical_and(ok, ei <= it)
                cv = cv + ok.astype(jnp.int32)
            return cv

        cv = lax.fori_loop(0, neblk, b, zero16i)
        return jnp.sum(cv)

    def eq_cond(st):
        lo_, hi_ = st
        return hi_ - lo_ > 1

    def eq_body(st):
        lo_, hi_ = st
        mid = lax.div(lo_ + hi_, 2)
        ce = count_eq_le(mid)
        lo_ = jnp.where(ce < m_eq, mid, lo_)
        hi_ = jnp.where(ce < m_eq, hi_, mid)
        return (lo_, hi_)

    total_eq = count_eq_le(jnp.int32(D_SAE))
    lo_init = jnp.where(total_eq == m_eq, jnp.int32(D_SAE - 1),
                        jnp.int32(-1))
    _, ithr = lax.while_loop(eq_cond, eq_body,
                             (lo_init, jnp.int32(D_SAE)))
    itf = jnp.full((NLANE,), ithr, jnp.int32)

    def fc_body(i, ptr):
        for k in range(NSTR):
            valid = ecnts[k] > i
            ei = eidx[pl.ds((i * NSTR + k) * NLANE, NLANE)]
            ev = plsc.load_gather(zrow, [ei], mask=valid)
            gtm = jnp.logical_and(ev > tf, valid)
            eqm = jnp.logical_and(jnp.logical_and(ev == tf, valid),
                                  ei <= itf)
            m = jnp.logical_or(gtm, eqm)
            plsc.store_compressed(fvalp.at[pl.ds(ptr, NLANE)], ev, mask=m)
            plsc.store_compressed(fidxp.at[pl.ds(ptr, NLANE)], ei, mask=m)
            ptr = ptr + plsc.all_reduce_population_count(m)[0]
        return ptr

    lax.fori_loop(0, neblk, fc_body, jnp.int32(0))

    for g in range(K // NLANE):
        fidx[pl.ds(g * NLANE, NLANE)] = fidxp[pl.ds(g * NLANE, NLANE)]
    gather = pltpu.async_copy(wdec_hbm.at[fidx], wrows, sem_g)

    def zz_body(j, _):
        zrow[pl.ds(j * NLANE, NLANE)] = zero16f
        return 0

    plsc.parallel_loop(0, NVREG // 8, carry=jnp.int32(0))(
        lambda j8, _: ([zrow.__setitem__(pl.ds((j8 * 8 + u) * NLANE, NLANE),
                                         zero16f) for u in range(8)], 0)[1])
    for g in range(K // NLANE):
        idxv = fidx[pl.ds(g * NLANE, NLANE)]
        valv = fvalp[pl.ds(g * NLANE, NLANE)]
        plsc.store_scatter(zrow, [idxv], valv)
    zout = pltpu.async_copy(zrow, zsp_hbm.at[r], sem_z)

    gather.wait()

    for g in range(K // NLANE):
        vv = fvalp[pl.ds(g * NLANE, NLANE)]
        for u in range(NLANE):
            svb[pl.ds((g * NLANE + u) * NLANE, NLANE)] = jnp.full(
                (NLANE,), vv[u])

    NCH = 8
    NU = D_IN // NCH // NLANE
    for ch in range(NCH):
        base = ch * (D_IN // NCH)

        def jbody(j2, accs):
            out = list(accs)
            for rr in range(2):
                j = j2 * 2 + rr
                sv = svb[pl.ds(j * NLANE, NLANE)]
                rows = [wrows[j, pl.ds(base + u * NLANE, NLANE)]
                        for u in range(NU)]
                out = [a + sv * rv for a, rv in zip(out, rows)]
            return tuple(out)

        accs = plsc.parallel_loop(0, K // 2, carry=(zero16f,) * NU)(jbody)
        for u, a in enumerate(accs):
            off = base + u * NLANE
            xrow[pl.ds(off, NLANE)] = a + bprev[pl.ds(off, NLANE)]

    pltpu.sync_copy(xrow, xhat_hbm.at[r])
    zout.wait()


def _topk_decode(z, maxs, W_dec, b_pre):
    mesh = plsc.VectorSubcoreMesh(core_axis_name="c", subcore_axis_name="s")
    f32 = jnp.float32
    kfn = functools.partial(
        pl.kernel,
        out_type=(
            jax.ShapeDtypeStruct((32, D_SAE), f32),
            jax.ShapeDtypeStruct((32, D_IN), f32),
        ),
        mesh=mesh,
        compiler_params=pltpu.CompilerParams(needs_layout_passes=False),
        scratch_types=[
            pltpu.VMEM((D_SAE,), f32),
            pltpu.VMEM((CAPS * NSTR * NLANE,), f32),
            pltpu.VMEM((ECAPS * NSTR * NLANE,), jnp.int32),
            pltpu.VMEM((K + NLANE,), f32),
            pltpu.VMEM((K + NLANE,), jnp.int32),
            pltpu.VMEM((K,), jnp.int32),
            pltpu.VMEM((K, D_IN), f32),
            pltpu.VMEM((D_IN,), f32),
            pltpu.VMEM((D_IN,), f32),
            pltpu.VMEM((K * NLANE,), f32),
            pltpu.VMEM((G, NLANE), f32),
            pltpu.SemaphoreType.DMA,
            pltpu.SemaphoreType.DMA,
        ],
    )(_sc_body)
    return kfn(z, maxs, W_dec, b_pre)


def kernel(x, b_pre, W_enc, b_enc, W_dec):
    z, maxs = _encode(x, b_pre, W_enc, b_enc)
    z_sparse, x_hat = _topk_decode(z, maxs, W_dec, b_pre)
    return (x_hat, z_sparse)
